# baseline (speedup 1.0000x reference)
import functools

import jax
import jax.numpy as jnp
from jax import lax
from jax.experimental import pallas as pl
from jax.experimental.pallas import tpu as pltpu
from jax.experimental.pallas import tpu_sc as plsc

BN_EPS = 1e-5
_NC = 2
_NS = 16
_NW = _NC * _NS
_CH = 128
_NP = 2


def _pad_edges(idx, fill, n_pass, k):
    e = idx.shape[0]
    tot = n_pass * _NW * k * _CH
    padb = jnp.full((tot - e,), fill, dtype=idx.dtype)
    return jnp.concatenate([idx, padb]).reshape(n_pass, _NW, k, _CH)


def _sc_scatter_add(x, src_c, dst_c):
    n, d = x.shape
    n_pass, nw, k, ch = src_c.shape
    assert nw == _NW and ch == _CH and k % 2 == 0
    rpt = (n // _NS) & ~7
    last = n - rpt * (_NS - 1)

    mesh = plsc.VectorSubcoreMesh(
        core_axis_name="c", subcore_axis_name="s", num_cores=_NC,
        num_subcores=_NS)

    scratch = [
        pltpu.VMEM((k, _CH), jnp.int32),
        pltpu.VMEM((k, _CH), jnp.int32),
        pltpu.VMEM((_CH, d), jnp.float32),
        pltpu.VMEM((_CH, d), jnp.float32),
        pltpu.VMEM_SHARED((n + 8, d), jnp.float32),
        pltpu.SemaphoreType.DMA,
        pltpu.SemaphoreType.DMA,
    ]

    @functools.partial(
        pl.kernel, mesh=mesh,
        out_type=jax.ShapeDtypeStruct((_NC, n, d), jnp.float32),
        scratch_types=scratch,
    )
    def sc_body(x_hbm, src_hbm, dst_hbm, out_hbm, si, di, rows0, rows1,
                agg, sem0, sem1):
        cid = lax.axis_index("c")
        sid = lax.axis_index("s")
        wid = sid * _NC + cid

        @pl.when(sid < _NS - 1)
        def _():
            r0 = sid * rpt
            pltpu.sync_copy(x_hbm.at[pl.ds(r0, rpt)], agg.at[pl.ds(r0, rpt)])

        @pl.when(sid == _NS - 1)
        def _():
            r0 = (_NS - 1) * rpt
            pltpu.sync_copy(x_hbm.at[pl.ds(r0, last)], agg.at[pl.ds(r0, last)])

        plsc.subcore_barrier()

        for p in range(n_pass):
            pltpu.sync_copy(src_hbm.at[p, wid], si)
            pltpu.sync_copy(dst_hbm.at[p, wid], di)

            pltpu.async_copy(x_hbm.at[si.at[0]], rows0, sem0)
            pltpu.async_copy(x_hbm.at[si.at[1]], rows1, sem1)

            def body(j2, carry):
                pltpu.make_async_copy(x_hbm.at[si.at[j2]], rows0, sem0).wait()
                pltpu.sync_copy(rows0, agg.at[di.at[j2]], add=True)
                pltpu.async_copy(x_hbm.at[si.at[j2 + 2]], rows0, sem0)
                pltpu.make_async_copy(
                    x_hbm.at[si.at[j2 + 1]], rows1, sem1).wait()
                pltpu.sync_copy(rows1, agg.at[di.at[j2 + 1]], add=True)
                pltpu.async_copy(x_hbm.at[si.at[j2 + 3]], rows1, sem1)
                return carry

            lax.fori_loop(0, (k - 2) // 2, lambda i, c: body(2 * i, c), 0,
                          unroll=False)

            j2 = k - 2
            pltpu.make_async_copy(x_hbm.at[si.at[j2]], rows0, sem0).wait()
            pltpu.sync_copy(rows0, agg.at[di.at[j2]], add=True)
            pltpu.make_async_copy(x_hbm.at[si.at[j2 + 1]], rows1, sem1).wait()
            pltpu.sync_copy(rows1, agg.at[di.at[j2 + 1]], add=True)

        plsc.subcore_barrier()

        @pl.when(sid < _NS - 1)
        def _():
            r0 = sid * rpt
            pltpu.sync_copy(agg.at[pl.ds(r0, rpt)],
                            out_hbm.at[cid, pl.ds(r0, rpt)])

        @pl.when(sid == _NS - 1)
        def _():
            r0 = (_NS - 1) * rpt
            pltpu.sync_copy(agg.at[pl.ds(r0, last)],
                            out_hbm.at[cid, pl.ds(r0, last)])

    return sc_body(x, src_c, dst_c)


def _mlp_bn_relu(h0, Wa, ba, Wb, bb, g, be):
    hp = jax.lax.Precision.HIGHEST
    h = jnp.dot(h0, Wa, precision=hp) + ba
    h = jnp.maximum(h, 0.0)
    h = jnp.dot(h, Wb, precision=hp) + bb
    mean = jnp.mean(h, axis=0, keepdims=True)
    var = jnp.mean((h - mean) ** 2, axis=0, keepdims=True)
    h = g * (h - mean) / jnp.sqrt(var + BN_EPS) + be
    return jnp.maximum(h, 0.0)


def _tc_layer(x, agg, Wa, ba, Wb, bb, g, be):
    n, d = x.shape
    h = Wa.shape[1]

    def body(x_ref, agg_ref, wa, ba_r, wb, bb_r, g_r, be_r, o_ref):
        h0 = agg_ref[0] + agg_ref[1] - x_ref[...]
        o_ref[...] = _mlp_bn_relu(h0, wa[...], ba_r[...], wb[...], bb_r[...],
                                  g_r[...], be_r[...])

    return pl.pallas_call(
        body,
        out_shape=jax.ShapeDtypeStruct((n, h), jnp.float32),
    )(x, agg, Wa, ba, Wb, bb, g, be)


def _tc_final(x, agg, batch, Wa, ba, Wb, bb, g, be, Wl, bl, num_graphs):
    n, d = x.shape
    out_dim = Wl.shape[1]

    def body(x_ref, agg_ref, batch_ref, wa, ba_r, wb, bb_r, g_r, be_r,
             wl, bl_r, o_ref):
        h0 = agg_ref[0] + agg_ref[1] - x_ref[...]
        h2 = _mlp_bn_relu(h0, wa[...], ba_r[...], wb[...], bb_r[...],
                          g_r[...], be_r[...])
        seg = batch_ref[...]
        gids = lax.broadcasted_iota(jnp.int32, (num_graphs, n), 0)
        ind = (seg[None, :] == gids).astype(jnp.float32)
        hp = jax.lax.Precision.HIGHEST
        pooled = jnp.dot(ind, h2, precision=hp)
        o_ref[...] = jnp.dot(pooled, wl[...], precision=hp) + bl_r[...]

    return pl.pallas_call(
        body,
        out_shape=jax.ShapeDtypeStruct((num_graphs, out_dim), jnp.float32),
    )(x, agg, batch, Wa, ba, Wb, bb, g, be, Wl, bl)


def kernel(x, edge_index, batch, W1a, b1a, W1b, b1b, g1, be1, W2a, b2a, W2b,
           b2b, g2, be2, Wl, bl):
    n = x.shape[0]
    e = edge_index.shape[1]
    num_graphs = 64

    k = -(-e // (_NP * _NW * _CH))
    if k % 2:
        k += 1
    src_c = _pad_edges(edge_index[0], 0, _NP, k)
    dst_c = _pad_edges(edge_index[1], n, _NP, k)

    agg1 = _sc_scatter_add(x, src_c, dst_c)
    h1 = _tc_layer(x, agg1, W1a, b1a, W1b, b1b, g1, be1)
    agg2 = _sc_scatter_add(h1, src_c, dst_c)
    out = _tc_final(h1, agg2, batch, W2a, b2a, W2b, b2b, g2, be2, Wl, bl,
                    num_graphs)
    return out


# speedup vs baseline: 1.0164x; 1.0164x over previous
import functools

import jax
import jax.numpy as jnp
from jax import lax
from jax.experimental import pallas as pl
from jax.experimental.pallas import tpu as pltpu
from jax.experimental.pallas import tpu_sc as plsc

BN_EPS = 1e-5
_NC = 2
_NS = 16
_NW = _NC * _NS
_CH = 128


def _pad_edges(idx, fill, n_chunks):
    e = idx.shape[0]
    per_w = e // _NW
    assert per_w * _NW == e
    pad = n_chunks * _CH - per_w
    body = idx.reshape(_NW, per_w)
    padb = jnp.full((_NW, pad), fill, dtype=idx.dtype)
    return jnp.concatenate([body, padb], axis=1)


def _sc_scatter_add(x, src_c, dst_c, n_chunks):
    n, d = x.shape
    nw = src_c.shape[0]
    assert nw == _NW and src_c.shape[1] == n_chunks * _CH
    assert n_chunks % 4 == 0 and n_chunks >= 8
    rpt = (n // _NS) & ~7
    last = n - rpt * (_NS - 1)

    mesh = plsc.VectorSubcoreMesh(
        core_axis_name="c", subcore_axis_name="s", num_cores=_NC,
        num_subcores=_NS)

    scratch = (
        [pltpu.VMEM((_CH,), jnp.int32) for _ in range(8)]
        + [pltpu.VMEM((_CH, d), jnp.float32) for _ in range(2)]
        + [pltpu.VMEM_SHARED((n + 8, d), jnp.float32)]
        + [pltpu.SemaphoreType.DMA for _ in range(6)]
    )

    @functools.partial(
        pl.kernel, mesh=mesh,
        out_type=jax.ShapeDtypeStruct((_NC, n, d), jnp.float32),
        scratch_types=scratch,
    )
    def sc_body(x_hbm, src_hbm, dst_hbm, out_hbm,
                si0, si1, si2, si3, di0, di1, di2, di3, rows0, rows1, agg,
                smi0, smi1, smi2, smi3, smg0, smg1):
        si = [si0, si1, si2, si3]
        di = [di0, di1, di2, di3]
        rows = [rows0, rows1]
        smi = [smi0, smi1, smi2, smi3]
        smg = [smg0, smg1]

        cid = lax.axis_index("c")
        sid = lax.axis_index("s")
        wid = sid * _NC + cid

        @pl.when(sid < _NS - 1)
        def _():
            r0 = sid * rpt
            pltpu.sync_copy(x_hbm.at[pl.ds(r0, rpt)], agg.at[pl.ds(r0, rpt)])

        @pl.when(sid == _NS - 1)
        def _():
            r0 = (_NS - 1) * rpt
            pltpu.sync_copy(x_hbm.at[pl.ds(r0, last)], agg.at[pl.ds(r0, last)])

        plsc.subcore_barrier()

        def fetch_idx(j, q):
            pltpu.async_copy(src_hbm.at[wid, pl.ds(j * _CH, _CH)], si[q],
                             smi[q])
            pltpu.async_copy(dst_hbm.at[wid, pl.ds(j * _CH, _CH)], di[q],
                             smi[q])

        def wait_idx(j, q):
            pltpu.make_async_copy(src_hbm.at[wid, pl.ds(0, _CH)], si[q],
                                  smi[q]).wait()
            pltpu.make_async_copy(dst_hbm.at[wid, pl.ds(0, _CH)], di[q],
                                  smi[q]).wait()

        def start_gather(q, r):
            pltpu.async_copy(x_hbm.at[si[q]], rows[r], smg[r])

        def wait_gather(q, r):
            pltpu.make_async_copy(x_hbm.at[si[q]], rows[r], smg[r]).wait()

        for q in range(4):
            fetch_idx(q, q)
        wait_idx(0, 0)
        start_gather(0, 0)
        wait_idx(1, 1)
        start_gather(1, 1)

        def turn(j, q, r, do_fetch, do_gather):
            wait_gather(q, r)
            pltpu.sync_copy(rows[r], agg.at[di[q]], add=True)
            if do_fetch:
                fetch_idx(j + 4, q)
            if do_gather:
                q2 = (q + 2) % 4
                wait_idx(j + 2, q2)
                start_gather(q2, r)

        def body(m, carry):
            j = m * 4
            turn(j, 0, 0, True, True)
            turn(j + 1, 1, 1, True, True)
            turn(j + 2, 2, 0, True, True)
            turn(j + 3, 3, 1, True, True)
            return carry

        n_full_m = (n_chunks - 4) // 4
        lax.fori_loop(0, n_full_m, body, 0, unroll=False)

        j = n_full_m * 4
        turn(j, 0, 0, False, True)
        turn(j + 1, 1, 1, False, True)
        turn(j + 2, 2, 0, False, False)
        turn(j + 3, 3, 1, False, False)

        plsc.subcore_barrier()

        @pl.when(sid < _NS - 1)
        def _():
            r0 = sid * rpt
            pltpu.sync_copy(agg.at[pl.ds(r0, rpt)],
                            out_hbm.at[cid, pl.ds(r0, rpt)])

        @pl.when(sid == _NS - 1)
        def _():
            r0 = (_NS - 1) * rpt
            pltpu.sync_copy(agg.at[pl.ds(r0, last)],
                            out_hbm.at[cid, pl.ds(r0, last)])

    return sc_body(x, src_c, dst_c)


def _mlp_bn_relu(h0, Wa, ba, Wb, bb, g, be):
    hp = jax.lax.Precision.HIGHEST
    h = jnp.dot(h0, Wa, precision=hp) + ba
    h = jnp.maximum(h, 0.0)
    h = jnp.dot(h, Wb, precision=hp) + bb
    mean = jnp.mean(h, axis=0, keepdims=True)
    var = jnp.mean((h - mean) ** 2, axis=0, keepdims=True)
    h = g * (h - mean) / jnp.sqrt(var + BN_EPS) + be
    return jnp.maximum(h, 0.0)


def _tc_layer(x, agg, Wa, ba, Wb, bb, g, be):
    n, d = x.shape
    h = Wa.shape[1]

    def body(x_ref, agg_ref, wa, ba_r, wb, bb_r, g_r, be_r, o_ref):
        h0 = agg_ref[0] + agg_ref[1] - x_ref[...]
        o_ref[...] = _mlp_bn_relu(h0, wa[...], ba_r[...], wb[...], bb_r[...],
                                  g_r[...], be_r[...])

    return pl.pallas_call(
        body,
        out_shape=jax.ShapeDtypeStruct((n, h), jnp.float32),
    )(x, agg, Wa, ba, Wb, bb, g, be)


def _tc_final(x, agg, batch, Wa, ba, Wb, bb, g, be, Wl, bl, num_graphs):
    n, d = x.shape
    out_dim = Wl.shape[1]

    def body(x_ref, agg_ref, batch_ref, wa, ba_r, wb, bb_r, g_r, be_r,
             wl, bl_r, o_ref):
        h0 = agg_ref[0] + agg_ref[1] - x_ref[...]
        h2 = _mlp_bn_relu(h0, wa[...], ba_r[...], wb[...], bb_r[...],
                          g_r[...], be_r[...])
        seg = batch_ref[...]
        gids = lax.broadcasted_iota(jnp.int32, (num_graphs, n), 0)
        ind = (seg[None, :] == gids).astype(jnp.float32)
        hp = jax.lax.Precision.HIGHEST
        pooled = jnp.dot(ind, h2, precision=hp)
        o_ref[...] = jnp.dot(pooled, wl[...], precision=hp) + bl_r[...]

    return pl.pallas_call(
        body,
        out_shape=jax.ShapeDtypeStruct((num_graphs, out_dim), jnp.float32),
    )(x, agg, batch, Wa, ba, Wb, bb, g, be, Wl, bl)


def kernel(x, edge_index, batch, W1a, b1a, W1b, b1b, g1, be1, W2a, b2a, W2b,
           b2b, g2, be2, Wl, bl):
    n = x.shape[0]
    e = edge_index.shape[1]
    num_graphs = 64

    n_chunks = -(-e // (_NW * _CH))
    n_chunks = -4 * (-n_chunks // 4)
    src_c = _pad_edges(edge_index[0], 0, n_chunks)
    dst_c = _pad_edges(edge_index[1], n, n_chunks)

    agg1 = _sc_scatter_add(x, src_c, dst_c, n_chunks)
    h1 = _tc_layer(x, agg1, W1a, b1a, W1b, b1b, g1, be1)
    agg2 = _sc_scatter_add(h1, src_c, dst_c, n_chunks)
    out = _tc_final(h1, agg2, batch, W2a, b2a, W2b, b2b, g2, be2, Wl, bl,
                    num_graphs)
    return out
